# baseline (speedup 1.0000x reference)
import functools

import jax
import jax.numpy as jnp
from jax import lax
from jax.experimental import pallas as pl
from jax.experimental.pallas import tpu as pltpu
from jax.experimental.pallas import tpu_sc as plsc

VOCAB = 100000
EMBED = 64
WINDOW = 200
NUM_SPEC = 10
BATCH = 1024

NC = 2
NS = 16
NW = NC * NS
ROWS_PER_W = BATCH // NW
RPB = 2
NBLK = ROWS_PER_W // RPB
NRING = 4
GRP_OFFS = [16 * g for g in range(WINDOW // 16)] + [WINDOW - 16]
NVEC = len(GRP_OFFS)
ROW_CHUNKS = [(0, 128), (128, WINDOW - 128)]


def _body(x_hbm, wpre_hbm, wspec_hbm, p_hbm, out_hbm,
          xs, buf0, buf1, buf2, buf3, p_vmem, spec_vmem,
          gsem0, gsem1, gsem2, gsem3, osem0, osem1, osem2, osem3):
  wid = lax.axis_index("s") * NC + lax.axis_index("c")
  base = wid * ROWS_PER_W

  pltpu.sync_copy(x_hbm.at[pl.ds(base, ROWS_PER_W)], xs)
  pltpu.sync_copy(p_hbm, p_vmem)
  pltpu.sync_copy(wspec_hbm, spec_vmem)

  bufs = (buf0, buf1, buf2, buf3)
  gsems = (gsem0, gsem1, gsem2, gsem3)
  osems = (osem0, osem1, osem2, osem3)

  def gather_copies(slot, b):
    cs = []
    for rr in range(RPB):
      r = RPB * b + rr
      for o, n in ROW_CHUNKS:
        cs.append(pltpu.make_async_copy(
            wpre_hbm.at[xs.at[r, pl.ds(o, n)]],
            bufs[slot].at[rr].at[pl.ds(o, n)], gsems[slot]))
    return cs

  def start_gather(slot, b):
    for c in gather_copies(slot, b):
      c.start()

  def wait_gather(slot, b):
    for c in gather_copies(slot, b):
      c.wait()

  def out_copy(slot, b):
    return pltpu.make_async_copy(
        bufs[slot], out_hbm.at[pl.ds(base + RPB * b, RPB)], osems[slot])

  def process(slot, b):
    buf = bufs[slot]

    def add_step(t, carry):
      for k in range(EMBED // 16):
        sl = pl.ds(16 * k, 16)
        pv = p_vmem[t, sl]
        for rr in range(RPB):
          buf[rr, t, sl] = buf[rr, t, sl] + pv
      return carry

    lax.fori_loop(0, WINDOW, add_step, 0, unroll=4)

    acc = jnp.zeros((16,), dtype=jnp.int32)
    for rr in range(RPB):
      for off in GRP_OFFS:
        ids = xs[RPB * b + rr, pl.ds(off, 16)]
        acc = acc | jnp.where(ids < NUM_SPEC, 1, 0).astype(jnp.int32)
    nspec = acc[0]
    for i in range(1, 16):
      nspec = nspec + acc[i]

    @pl.when(nspec > 0)
    def _patch():
      def patch_step(g, carry):
        off = pl.multiple_of(jnp.where(g == NVEC - 1, WINDOW - 16, 16 * g), 8)
        for rr in range(RPB):
          ids = xs[RPB * b + rr, pl.ds(off, 16)]
          for i in range(16):
            s = ids[i]
            t = off + i

            @pl.when(s < NUM_SPEC)
            def _fix():
              for k in range(EMBED // 16):
                sl = pl.ds(16 * k, 16)
                buf[rr, t, sl] = spec_vmem[s, sl] + p_vmem[t, sl]

        return carry

      lax.fori_loop(0, NVEC, patch_step, 0)

  start_gather(0, 0)
  start_gather(1, 1)

  def ring_step(j, carry):
    for q in range(NRING):
      b = NRING * j + q
      wait_gather(q, b)
      process(q, b)
      out_copy(q, b).start()

      ns = (q + 2) % NRING

      @pl.when(b + 2 < NBLK)
      def _prefetch():
        @pl.when(b >= 2)
        def _drain():
          out_copy(ns, b - 2).wait()

        start_gather(ns, b + 2)

    return carry

  lax.fori_loop(0, NBLK // NRING, ring_step, 0)

  for q in range(NRING):
    out_copy(q, NBLK - NRING + q).wait()


@jax.jit
def _run(x, W_pre, W_spec, P):
  mesh = plsc.VectorSubcoreMesh(core_axis_name="c", subcore_axis_name="s")
  f = pl.kernel(
      _body,
      out_type=jax.ShapeDtypeStruct((BATCH, WINDOW, EMBED), jnp.float32),
      mesh=mesh,
      scratch_types=[
          pltpu.VMEM((ROWS_PER_W, WINDOW), jnp.int32),
          pltpu.VMEM((RPB, WINDOW, EMBED), jnp.float32),
          pltpu.VMEM((RPB, WINDOW, EMBED), jnp.float32),
          pltpu.VMEM((RPB, WINDOW, EMBED), jnp.float32),
          pltpu.VMEM((RPB, WINDOW, EMBED), jnp.float32),
          pltpu.VMEM((WINDOW, EMBED), jnp.float32),
          pltpu.VMEM((NUM_SPEC, EMBED), jnp.float32),
          pltpu.SemaphoreType.DMA,
          pltpu.SemaphoreType.DMA,
          pltpu.SemaphoreType.DMA,
          pltpu.SemaphoreType.DMA,
          pltpu.SemaphoreType.DMA,
          pltpu.SemaphoreType.DMA,
          pltpu.SemaphoreType.DMA,
          pltpu.SemaphoreType.DMA,
      ],
      compiler_params=pltpu.CompilerParams(use_tc_tiling_on_sc=False),
  )
  return f(x, W_pre, W_spec, P)


def kernel(x, W_pre, W_spec, P):
  return _run(x.astype(jnp.int32), W_pre, W_spec, P)


# speedup vs baseline: 1.0593x; 1.0593x over previous
import functools

import jax
import jax.numpy as jnp
from jax import lax
from jax.experimental import pallas as pl
from jax.experimental.pallas import tpu as pltpu
from jax.experimental.pallas import tpu_sc as plsc

VOCAB = 100000
EMBED = 64
WINDOW = 200
NUM_SPEC = 10
BATCH = 1024

NC = 2
NS = 16
NW = NC * NS
ROWS_PER_W = BATCH // NW
RPB = 4
NBLK = ROWS_PER_W // RPB
GRP_OFFS = [16 * g for g in range(WINDOW // 16)] + [WINDOW - 16]
NVEC = len(GRP_OFFS)
ROW_CHUNKS = [(0, 128), (128, WINDOW - 128)]


def _body(x_hbm, wpre_hbm, wspec_hbm, p_hbm, out_hbm,
          xs, buf0, buf1, p_shared, p_vmem, spec_vmem,
          gsem0, gsem1, osem0, osem1):
  wid = lax.axis_index("s") * NC + lax.axis_index("c")
  base = wid * ROWS_PER_W

  pltpu.sync_copy(x_hbm.at[pl.ds(base, ROWS_PER_W)], xs)
  pltpu.sync_copy(p_hbm, p_vmem)
  pltpu.sync_copy(wspec_hbm, spec_vmem)

  @pl.when(lax.axis_index("s") == 0)
  def _stage_p():
    for rr in range(RPB):
      pltpu.sync_copy(p_hbm, p_shared.at[rr])

  plsc.subcore_barrier()

  bufs = (buf0, buf1)
  gsems = (gsem0, gsem1)
  osems = (osem0, osem1)

  def gather_copies(slot, b):
    cs = []
    for rr in range(RPB):
      r = RPB * b + rr
      for o, n in ROW_CHUNKS:
        cs.append(pltpu.make_async_copy(
            wpre_hbm.at[xs.at[r, pl.ds(o, n)]],
            bufs[slot].at[rr].at[pl.ds(o, n)], gsems[slot]))
    return cs

  def init_and_gather(slot, b):
    pltpu.sync_copy(p_shared, bufs[slot])
    for c in gather_copies(slot, b):
      c.start(add=True)

  def wait_gather(slot, b):
    for c in gather_copies(slot, b):
      c.wait()

  def out_copy(slot, b):
    return pltpu.make_async_copy(
        bufs[slot], out_hbm.at[pl.ds(base + RPB * b, RPB)], osems[slot])

  def process(slot, b):
    buf = bufs[slot]

    acc = jnp.zeros((16,), dtype=jnp.int32)
    for rr in range(RPB):
      for off in GRP_OFFS:
        ids = xs[RPB * b + rr, pl.ds(off, 16)]
        acc = acc | jnp.where(ids < NUM_SPEC, 1, 0).astype(jnp.int32)
    nspec = acc[0]
    for i in range(1, 16):
      nspec = nspec + acc[i]

    @pl.when(nspec > 0)
    def _patch():
      def patch_step(g, carry):
        off = pl.multiple_of(jnp.where(g == NVEC - 1, WINDOW - 16, 16 * g), 8)
        for rr in range(RPB):
          ids = xs[RPB * b + rr, pl.ds(off, 16)]
          for i in range(16):
            s = ids[i]
            t = off + i

            @pl.when(s < NUM_SPEC)
            def _fix():
              for k in range(EMBED // 16):
                sl = pl.ds(16 * k, 16)
                buf[rr, t, sl] = spec_vmem[s, sl] + p_vmem[t, sl]

        return carry

      lax.fori_loop(0, NVEC, patch_step, 0)

  init_and_gather(0, 0)

  def pair_step(i, carry):
    ba = 2 * i
    bb = 2 * i + 1

    @pl.when(i > 0)
    def _drain1():
      out_copy(1, bb - 2).wait()

    init_and_gather(1, bb)

    wait_gather(0, ba)
    process(0, ba)
    out_copy(0, ba).start()

    wait_gather(1, bb)
    process(1, bb)
    out_copy(1, bb).start()

    @pl.when(i < NBLK // 2 - 1)
    def _next0():
      out_copy(0, ba).wait()
      init_and_gather(0, ba + 2)

    return carry

  lax.fori_loop(0, NBLK // 2, pair_step, 0)

  out_copy(0, NBLK - 2).wait()
  out_copy(1, NBLK - 1).wait()


@jax.jit
def _run(x, W_pre, W_spec, P):
  mesh = plsc.VectorSubcoreMesh(core_axis_name="c", subcore_axis_name="s")
  f = pl.kernel(
      _body,
      out_type=jax.ShapeDtypeStruct((BATCH, WINDOW, EMBED), jnp.float32),
      mesh=mesh,
      scratch_types=[
          pltpu.VMEM((ROWS_PER_W, WINDOW), jnp.int32),
          pltpu.VMEM((RPB, WINDOW, EMBED), jnp.float32),
          pltpu.VMEM((RPB, WINDOW, EMBED), jnp.float32),
          pltpu.VMEM_SHARED((RPB, WINDOW, EMBED), jnp.float32),
          pltpu.VMEM((WINDOW, EMBED), jnp.float32),
          pltpu.VMEM((NUM_SPEC, EMBED), jnp.float32),
          pltpu.SemaphoreType.DMA,
          pltpu.SemaphoreType.DMA,
          pltpu.SemaphoreType.DMA,
          pltpu.SemaphoreType.DMA,
      ],
      compiler_params=pltpu.CompilerParams(use_tc_tiling_on_sc=False),
  )
  return f(x, W_pre, W_spec, P)


def kernel(x, W_pre, W_spec, P):
  return _run(x.astype(jnp.int32), W_pre, W_spec, P)
